# Initial kernel scaffold; baseline (speedup 1.0000x reference)
import jax
import jax.numpy as jnp
from jax.experimental import pallas as pl


def kernel(x, edge_weight, W_ih, W_hh, b_ih, b_hh, gamma, beta, edge_index):
    raise NotImplementedError("write your pallas kernel here")


# speedup vs baseline: 3.5367x; 3.5367x over previous
import functools

import jax
import jax.numpy as jnp
from jax import lax
from jax.experimental import pallas as pl
from jax.experimental.pallas import tpu as pltpu
from jax.experimental.pallas import tpu_sc as plsc

N_NODES = 10000
N_PAD = 10240
D = 128
H = 128
NUM_CORES = 2
NUM_SUBCORES = 16
NW = NUM_CORES * NUM_SUBCORES
CHUNK = 128
LANES = 16
RPT = N_PAD // NUM_SUBCORES


def _sc_body(x_hbm, src_hbm, dst_hbm, w_hbm, zeros_hbm, out_hbm,
             src_v, dst_v, w_v, rows_v, acc_sh, sem):
    num_hops = src_hbm.shape[0]
    num_chunks = src_hbm.shape[2]
    cid = lax.axis_index("c")
    sid = lax.axis_index("s")
    wid = cid * NUM_SUBCORES + sid

    pltpu.sync_copy(zeros_hbm, acc_sh.at[pl.ds(sid * RPT, RPT)])
    plsc.subcore_barrier()

    for hop in range(num_hops):
        pltpu.sync_copy(src_hbm.at[hop, wid], src_v)
        pltpu.sync_copy(dst_hbm.at[hop, wid], dst_v)
        pltpu.sync_copy(w_hbm.at[hop, wid], w_v)

        def chunk_body(k, carry):
            pltpu.async_copy(x_hbm.at[src_v.at[k]], rows_v, sem).wait()

            def grp_body(g, c2):
                wvec = w_v[k, pl.ds(g * LANES, LANES)]
                for l in range(LANES):
                    wt = wvec[l]
                    e = g * LANES + l
                    for j in range(D // LANES):
                        sl = pl.ds(j * LANES, LANES)
                        rows_v[e, sl] = rows_v[e, sl] * wt
                return c2

            lax.fori_loop(0, CHUNK // LANES, grp_body, 0)
            pltpu.sync_copy(rows_v, acc_sh.at[dst_v.at[k]], add=True)
            return carry

        lax.fori_loop(0, num_chunks, chunk_body, 0)
        plsc.subcore_barrier()
        pltpu.sync_copy(acc_sh.at[pl.ds(sid * RPT, RPT)],
                        out_hbm.at[hop, cid, pl.ds(sid * RPT, RPT)])
        plsc.subcore_barrier()


def _sc_spmm(x, src, dst, w):
    num_hops, _, num_chunks, _ = src.shape
    zeros = jnp.zeros((RPT, D), jnp.float32)
    mesh = plsc.VectorSubcoreMesh(core_axis_name="c", subcore_axis_name="s")
    f = pl.kernel(
        _sc_body,
        out_type=jax.ShapeDtypeStruct((num_hops, NUM_CORES, N_PAD, D),
                                      jnp.float32),
        mesh=mesh,
        scratch_types=[
            pltpu.VMEM((num_chunks, CHUNK), jnp.int32),
            pltpu.VMEM((num_chunks, CHUNK), jnp.int32),
            pltpu.VMEM((num_chunks, CHUNK), jnp.float32),
            pltpu.VMEM((CHUNK, D), jnp.float32),
            pltpu.VMEM_SHARED((N_PAD, D), jnp.float32),
            pltpu.SemaphoreType.DMA,
        ],
    )
    return f(x, src, dst, w, zeros)


def _tc_body(p_ref, wih_ref, whh_ref, bih_ref, bhh_ref, g_ref, b_ref, o_ref):
    num_hops = p_ref.shape[0]
    bn = o_ref.shape[0]
    h = jnp.zeros((bn, H), jnp.float32)
    acc = jnp.zeros((bn, H), jnp.float32)
    for c in range(num_hops):
        hx = jnp.maximum(p_ref[c, 0] + p_ref[c, 1], 0.0)
        gi = jnp.dot(hx, wih_ref[...], preferred_element_type=jnp.float32)
        gi = gi + bih_ref[...]
        gh = jnp.dot(h, whh_ref[...], preferred_element_type=jnp.float32)
        gh = gh + bhh_ref[...]
        r = jax.nn.sigmoid(gi[:, :H] + gh[:, :H])
        z = jax.nn.sigmoid(gi[:, H:2 * H] + gh[:, H:2 * H])
        n = jnp.tanh(gi[:, 2 * H:] + r * gh[:, 2 * H:])
        h = (1.0 - z) * n + z * h
        acc = acc + h
    mean = jnp.mean(acc, axis=-1, keepdims=True)
    var = jnp.mean((acc - mean) ** 2, axis=-1, keepdims=True)
    o_ref[...] = (acc - mean) * lax.rsqrt(var + 1e-5) * g_ref[...] + b_ref[...]


def _tc_gru(partials, W_ihT, W_hhT, b_ih, b_hh, gamma, beta, interpret=False):
    num_hops = partials.shape[0]
    bn = 1024
    grid = (N_PAD // bn,)
    return pl.pallas_call(
        _tc_body,
        grid=grid,
        in_specs=[
            pl.BlockSpec((num_hops, NUM_CORES, bn, D),
                         lambda i: (0, 0, i, 0)),
            pl.BlockSpec((D, 3 * H), lambda i: (0, 0)),
            pl.BlockSpec((H, 3 * H), lambda i: (0, 0)),
            pl.BlockSpec((1, 3 * H), lambda i: (0, 0)),
            pl.BlockSpec((1, 3 * H), lambda i: (0, 0)),
            pl.BlockSpec((1, H), lambda i: (0, 0)),
            pl.BlockSpec((1, H), lambda i: (0, 0)),
        ],
        out_specs=pl.BlockSpec((bn, H), lambda i: (i, 0)),
        out_shape=jax.ShapeDtypeStruct((N_PAD, H), jnp.float32),
        interpret=interpret,
    )(partials, W_ihT, W_hhT, b_ih.reshape(1, -1), b_hh.reshape(1, -1),
      gamma.reshape(1, -1), beta.reshape(1, -1))


def kernel(x, edge_weight, W_ih, W_hh, b_ih, b_hh, gamma, beta, edge_index):
    num_hops, _, num_edges = edge_index.shape
    grp = NW * CHUNK
    e_pad = ((num_edges + grp - 1) // grp) * grp
    pad = e_pad - num_edges
    dst = jnp.pad(edge_index[:, 0, :], ((0, 0), (0, pad)))
    src = jnp.pad(edge_index[:, 1, :], ((0, 0), (0, pad)))
    w = jnp.pad(edge_weight, ((0, 0), (0, pad)))
    num_chunks = e_pad // grp
    dst = dst.reshape(num_hops, NW, num_chunks, CHUNK)
    src = src.reshape(num_hops, NW, num_chunks, CHUNK)
    w = w.reshape(num_hops, NW, num_chunks, CHUNK)

    partials = _sc_spmm(x, src, dst, w)
    out = _tc_gru(partials, W_ih.T, W_hh.T, b_ih, b_hh, gamma, beta)
    return out[:N_NODES]


# speedup vs baseline: 5.7541x; 1.6270x over previous
import jax
import jax.numpy as jnp
from jax import lax
from jax.experimental import pallas as pl
from jax.experimental.pallas import tpu as pltpu
from jax.experimental.pallas import tpu_sc as plsc

N_NODES = 10000
N_PAD = 10240
D = 128
H = 128
NUM_CORES = 2
NUM_SUBCORES = 16
NW = NUM_CORES * NUM_SUBCORES
CHUNK = 128
LANES = 16
RPT = N_PAD // NUM_SUBCORES


def _scale_chunk(w_v, rows, k):
    def grp_body(g, c2):
        wvec = w_v[k, pl.ds(g * LANES, LANES)]
        for l in range(LANES):
            wt = wvec[l]
            e = g * LANES + l
            for j in range(D // LANES):
                sl = pl.ds(j * LANES, LANES)
                rows[e, sl] = rows[e, sl] * wt
        return c2

    lax.fori_loop(0, CHUNK // LANES, grp_body, 0)


def _sc_body(x_hbm, src_hbm, dst_hbm, w_hbm, zeros_hbm, out_hbm,
             src_v, dst_v, w_v, rows_v, acc_sh, sem):
    num_hops = src_hbm.shape[0]
    num_chunks = src_hbm.shape[2]
    cid = lax.axis_index("c")
    sid = lax.axis_index("s")
    wid = cid * NUM_SUBCORES + sid

    pltpu.sync_copy(zeros_hbm, acc_sh.at[pl.ds(sid * RPT, RPT)])
    plsc.subcore_barrier()

    for hop in range(num_hops):
        pltpu.sync_copy(src_hbm.at[hop, wid], src_v)
        pltpu.sync_copy(dst_hbm.at[hop, wid], dst_v)
        pltpu.sync_copy(w_hbm.at[hop, wid], w_v)
        pltpu.async_copy(x_hbm.at[src_v.at[0]], rows_v, sem)

        def chunk_body(k, carry):
            pltpu.make_async_copy(x_hbm.at[src_v.at[k]], rows_v, sem).wait()
            _scale_chunk(w_v, rows_v, k)
            pltpu.sync_copy(rows_v, acc_sh.at[dst_v.at[k]], add=True)

            @pl.when(k + 1 < num_chunks)
            def _():
                pltpu.async_copy(x_hbm.at[src_v.at[k + 1]], rows_v, sem)

            return carry

        lax.fori_loop(0, num_chunks, chunk_body, 0)
        plsc.subcore_barrier()
        pltpu.sync_copy(acc_sh.at[pl.ds(sid * RPT, RPT)],
                        out_hbm.at[hop, cid, pl.ds(sid * RPT, RPT)])
        plsc.subcore_barrier()


def _sc_spmm(x, src, dst, w):
    num_hops, _, num_chunks, _ = src.shape
    zeros = jnp.zeros((RPT, D), jnp.float32)
    mesh = plsc.VectorSubcoreMesh(core_axis_name="c", subcore_axis_name="s")
    f = pl.kernel(
        _sc_body,
        out_type=jax.ShapeDtypeStruct((num_hops, NUM_CORES, N_PAD, D),
                                      jnp.float32),
        mesh=mesh,
        scratch_types=[
            pltpu.VMEM((num_chunks, CHUNK), jnp.int32),
            pltpu.VMEM((num_chunks, CHUNK), jnp.int32),
            pltpu.VMEM((num_chunks, CHUNK), jnp.float32),
            pltpu.VMEM((CHUNK, D), jnp.float32),
            pltpu.VMEM_SHARED((N_PAD, D), jnp.float32),
            pltpu.SemaphoreType.DMA,
        ],
    )
    return f(x, src, dst, w, zeros)


def _tc_body(p_ref, wih_ref, whh_ref, bih_ref, bhh_ref, g_ref, b_ref, o_ref):
    num_hops = p_ref.shape[0]
    bn = o_ref.shape[0]
    h = jnp.zeros((bn, H), jnp.float32)
    acc = jnp.zeros((bn, H), jnp.float32)
    for c in range(num_hops):
        hx = jnp.maximum(p_ref[c, 0] + p_ref[c, 1], 0.0)
        gi = jnp.dot(hx, wih_ref[...], preferred_element_type=jnp.float32)
        gi = gi + bih_ref[...]
        gh = jnp.dot(h, whh_ref[...], preferred_element_type=jnp.float32)
        gh = gh + bhh_ref[...]
        r = jax.nn.sigmoid(gi[:, :H] + gh[:, :H])
        z = jax.nn.sigmoid(gi[:, H:2 * H] + gh[:, H:2 * H])
        n = jnp.tanh(gi[:, 2 * H:] + r * gh[:, 2 * H:])
        h = (1.0 - z) * n + z * h
        acc = acc + h
    mean = jnp.mean(acc, axis=-1, keepdims=True)
    var = jnp.mean((acc - mean) ** 2, axis=-1, keepdims=True)
    o_ref[...] = (acc - mean) * lax.rsqrt(var + 1e-5) * g_ref[...] + b_ref[...]


def _tc_gru(partials, W_ihT, W_hhT, b_ih, b_hh, gamma, beta, interpret=False):
    num_hops = partials.shape[0]
    bn = 1024
    grid = (N_PAD // bn,)
    return pl.pallas_call(
        _tc_body,
        grid=grid,
        in_specs=[
            pl.BlockSpec((num_hops, NUM_CORES, bn, D),
                         lambda i: (0, 0, i, 0)),
            pl.BlockSpec((D, 3 * H), lambda i: (0, 0)),
            pl.BlockSpec((H, 3 * H), lambda i: (0, 0)),
            pl.BlockSpec((1, 3 * H), lambda i: (0, 0)),
            pl.BlockSpec((1, 3 * H), lambda i: (0, 0)),
            pl.BlockSpec((1, H), lambda i: (0, 0)),
            pl.BlockSpec((1, H), lambda i: (0, 0)),
        ],
        out_specs=pl.BlockSpec((bn, H), lambda i: (i, 0)),
        out_shape=jax.ShapeDtypeStruct((N_PAD, H), jnp.float32),
        interpret=interpret,
    )(partials, W_ihT, W_hhT, b_ih.reshape(1, -1), b_hh.reshape(1, -1),
      gamma.reshape(1, -1), beta.reshape(1, -1))


def kernel(x, edge_weight, W_ih, W_hh, b_ih, b_hh, gamma, beta, edge_index):
    num_hops, _, num_edges = edge_index.shape
    grp = NW * CHUNK
    e_pad = ((num_edges + grp - 1) // grp) * grp
    pad = e_pad - num_edges
    pad_idx = jnp.arange(pad, dtype=jnp.int32)
    dst = jnp.concatenate(
        [edge_index[:, 0, :],
         jnp.broadcast_to(pad_idx % N_PAD, (num_hops, pad))], axis=1)
    src = jnp.concatenate(
        [edge_index[:, 1, :],
         jnp.broadcast_to(pad_idx % N_NODES, (num_hops, pad))], axis=1)
    w = jnp.pad(edge_weight, ((0, 0), (0, pad)))
    num_chunks = e_pad // (NW * CHUNK)
    dst = dst.reshape(num_hops, NW, num_chunks, CHUNK)
    src = src.reshape(num_hops, NW, num_chunks, CHUNK)
    w = w.reshape(num_hops, NW, num_chunks, CHUNK)

    partials = _sc_spmm(x, src, dst, w)
    out = _tc_gru(partials, W_ih.T, W_hh.T, b_ih, b_hh, gamma, beta)
    return out[:N_NODES]


# speedup vs baseline: 8.5372x; 1.4837x over previous
import jax
import jax.numpy as jnp
from jax import lax
from jax.experimental import pallas as pl
from jax.experimental.pallas import tpu as pltpu
from jax.experimental.pallas import tpu_sc as plsc

N_NODES = 10000
N_PAD = 10240
D = 128
H = 128
NUM_CORES = 2
NUM_SUBCORES = 16
NW = NUM_CORES * NUM_SUBCORES
CHUNK = 128
LANES = 16
SLAB = 16
RPT = N_PAD // NUM_SUBCORES


def _scale_chunk(w_v, rows, k):
    def grp_body(g, c2):
        wvec = w_v[k, pl.ds(g * LANES, LANES)]
        for l in range(LANES):
            wt = wvec[l]
            e = g * LANES + l
            for j in range(D // LANES):
                sl = pl.ds(j * LANES, LANES)
                rows[e, sl] = rows[e, sl] * wt
        return c2

    lax.fori_loop(0, CHUNK // LANES, grp_body, 0)


def _sc_body(x_hbm, src_hbm, dst_hbm, w_hbm, zeros_hbm, out_hbm,
             src_v, dst_v, w_v, rows0, rows1, acc_sh,
             gsem0, gsem1, ssem0, ssem1):
    rows = (rows0, rows1)
    gsems = (gsem0, gsem1)
    ssems = (ssem0, ssem1)
    num_hops = src_hbm.shape[0]
    num_chunks = src_hbm.shape[2]
    num_slabs = num_chunks // SLAB
    nq = SLAB // 2
    cid = lax.axis_index("c")
    sid = lax.axis_index("s")
    wid = cid * NUM_SUBCORES + sid

    def drain_scatter(b):
        pltpu.make_async_copy(zeros_hbm.at[pl.ds(0, CHUNK)], rows[b],
                              ssems[b]).wait()

    pltpu.sync_copy(zeros_hbm, acc_sh.at[pl.ds(sid * RPT, RPT)])
    plsc.subcore_barrier()

    for hop in range(num_hops):

        def slab_body(s, carry):
            sl = pl.ds(s * SLAB, SLAB)
            pltpu.sync_copy(src_hbm.at[hop, wid, sl], src_v)
            pltpu.sync_copy(dst_hbm.at[hop, wid, sl], dst_v)
            pltpu.sync_copy(w_hbm.at[hop, wid, sl], w_v)
            pltpu.async_copy(x_hbm.at[src_v.at[0]], rows[0], gsems[0])

            def qbody(q, c2):
                for b in range(2):
                    j = q * 2 + b
                    pltpu.make_async_copy(x_hbm.at[src_v.at[j]], rows[b],
                                          gsems[b]).wait()
                    if b == 1:
                        drain_scatter(0)
                        @pl.when(q < nq - 1)
                        def _():
                            pltpu.async_copy(x_hbm.at[src_v.at[j + 1]],
                                             rows[0], gsems[0])
                    else:
                        @pl.when(q > 0)
                        def _():
                            drain_scatter(1)
                        pltpu.async_copy(x_hbm.at[src_v.at[j + 1]],
                                         rows[1], gsems[1])
                    _scale_chunk(w_v, rows[b], j)
                    pltpu.async_copy(rows[b], acc_sh.at[dst_v.at[j]],
                                     ssems[b], add=True)
                return c2

            lax.fori_loop(0, nq, qbody, 0)
            drain_scatter(1)
            return carry

        lax.fori_loop(0, num_slabs, slab_body, 0)
        plsc.subcore_barrier()
        pltpu.sync_copy(acc_sh.at[pl.ds(sid * RPT, RPT)],
                        out_hbm.at[hop, cid, pl.ds(sid * RPT, RPT)])
        plsc.subcore_barrier()


def _sc_spmm(x, src, dst, w):
    num_hops, _, num_chunks, _ = src.shape
    zeros = jnp.zeros((RPT, D), jnp.float32)
    mesh = plsc.VectorSubcoreMesh(core_axis_name="c", subcore_axis_name="s")
    f = pl.kernel(
        _sc_body,
        out_type=jax.ShapeDtypeStruct((num_hops, NUM_CORES, N_PAD, D),
                                      jnp.float32),
        mesh=mesh,
        scratch_types=[
            pltpu.VMEM((SLAB, CHUNK), jnp.int32),
            pltpu.VMEM((SLAB, CHUNK), jnp.int32),
            pltpu.VMEM((SLAB, CHUNK), jnp.float32),
            pltpu.VMEM((CHUNK, D), jnp.float32),
            pltpu.VMEM((CHUNK, D), jnp.float32),
            pltpu.VMEM_SHARED((N_PAD, D), jnp.float32),
            pltpu.SemaphoreType.DMA,
            pltpu.SemaphoreType.DMA,
            pltpu.SemaphoreType.DMA,
            pltpu.SemaphoreType.DMA,
        ],
    )
    return f(x, src, dst, w, zeros)


def _tc_body(p_ref, wih_ref, whh_ref, bih_ref, bhh_ref, g_ref, b_ref, o_ref):
    num_hops = p_ref.shape[0]
    bn = o_ref.shape[0]
    h = jnp.zeros((bn, H), jnp.float32)
    acc = jnp.zeros((bn, H), jnp.float32)
    for c in range(num_hops):
        hx = jnp.maximum(p_ref[c, 0] + p_ref[c, 1], 0.0)
        gi = jnp.dot(hx, wih_ref[...], preferred_element_type=jnp.float32)
        gi = gi + bih_ref[...]
        gh = jnp.dot(h, whh_ref[...], preferred_element_type=jnp.float32)
        gh = gh + bhh_ref[...]
        r = jax.nn.sigmoid(gi[:, :H] + gh[:, :H])
        z = jax.nn.sigmoid(gi[:, H:2 * H] + gh[:, H:2 * H])
        n = jnp.tanh(gi[:, 2 * H:] + r * gh[:, 2 * H:])
        h = (1.0 - z) * n + z * h
        acc = acc + h
    mean = jnp.mean(acc, axis=-1, keepdims=True)
    var = jnp.mean((acc - mean) ** 2, axis=-1, keepdims=True)
    o_ref[...] = (acc - mean) * lax.rsqrt(var + 1e-5) * g_ref[...] + b_ref[...]


def _tc_gru(partials, W_ihT, W_hhT, b_ih, b_hh, gamma, beta, interpret=False):
    num_hops = partials.shape[0]
    bn = 1024
    grid = (N_PAD // bn,)
    return pl.pallas_call(
        _tc_body,
        grid=grid,
        in_specs=[
            pl.BlockSpec((num_hops, NUM_CORES, bn, D),
                         lambda i: (0, 0, i, 0)),
            pl.BlockSpec((D, 3 * H), lambda i: (0, 0)),
            pl.BlockSpec((H, 3 * H), lambda i: (0, 0)),
            pl.BlockSpec((1, 3 * H), lambda i: (0, 0)),
            pl.BlockSpec((1, 3 * H), lambda i: (0, 0)),
            pl.BlockSpec((1, H), lambda i: (0, 0)),
            pl.BlockSpec((1, H), lambda i: (0, 0)),
        ],
        out_specs=pl.BlockSpec((bn, H), lambda i: (i, 0)),
        out_shape=jax.ShapeDtypeStruct((N_PAD, H), jnp.float32),
        interpret=interpret,
    )(partials, W_ihT, W_hhT, b_ih.reshape(1, -1), b_hh.reshape(1, -1),
      gamma.reshape(1, -1), beta.reshape(1, -1))


def kernel(x, edge_weight, W_ih, W_hh, b_ih, b_hh, gamma, beta, edge_index):
    num_hops, _, num_edges = edge_index.shape
    grp = NW * CHUNK * SLAB
    e_pad = ((num_edges + grp - 1) // grp) * grp
    pad = e_pad - num_edges
    pad_idx = jnp.arange(pad, dtype=jnp.int32)
    dst = jnp.concatenate(
        [edge_index[:, 0, :],
         jnp.broadcast_to(pad_idx % N_PAD, (num_hops, pad))], axis=1)
    src = jnp.concatenate(
        [edge_index[:, 1, :],
         jnp.broadcast_to(pad_idx % N_NODES, (num_hops, pad))], axis=1)
    w = jnp.pad(edge_weight, ((0, 0), (0, pad)))
    num_chunks = e_pad // (NW * CHUNK)
    dst = dst.reshape(num_hops, NW, num_chunks, CHUNK)
    src = src.reshape(num_hops, NW, num_chunks, CHUNK)
    w = w.reshape(num_hops, NW, num_chunks, CHUNK)

    partials = _sc_spmm(x, src, dst, w)
    out = _tc_gru(partials, W_ih.T, W_hh.T, b_ih, b_hh, gamma, beta)
    return out[:N_NODES]


# speedup vs baseline: 8.9893x; 1.0530x over previous
import jax
import jax.numpy as jnp
from jax import lax
from jax.experimental import pallas as pl
from jax.experimental.pallas import tpu as pltpu
from jax.experimental.pallas import tpu_sc as plsc

N_NODES = 10000
N_PAD = 10240
D = 128
H = 128
NUM_CORES = 2
NUM_SUBCORES = 16
NW = NUM_CORES * NUM_SUBCORES
CHUNK = 64
LANES = 16
NBUF = 4
SLAB = 32
RPT = N_PAD // NUM_SUBCORES


def _scale_chunk(w_v, rows, k):
    def grp_body(g, c2):
        wvec = w_v[k, pl.ds(g * LANES, LANES)]
        for l in range(LANES):
            wt = wvec[l]
            e = g * LANES + l
            for j in range(D // LANES):
                sl = pl.ds(j * LANES, LANES)
                rows[e, sl] = rows[e, sl] * wt
        return c2

    lax.fori_loop(0, CHUNK // LANES, grp_body, 0)


def _sc_body(x_hbm, src_hbm, dst_hbm, w_hbm, zeros_hbm, out_hbm,
             src_v, dst_v, w_v, rows0, rows1, rows2, rows3, acc_sh,
             gsem0, gsem1, gsem2, gsem3, ssem0, ssem1, ssem2, ssem3):
    rows = (rows0, rows1, rows2, rows3)
    gsems = (gsem0, gsem1, gsem2, gsem3)
    ssems = (ssem0, ssem1, ssem2, ssem3)
    num_hops = src_hbm.shape[0]
    num_chunks = src_hbm.shape[2]
    num_slabs = num_chunks // SLAB
    nq = SLAB // NBUF
    cid = lax.axis_index("c")
    sid = lax.axis_index("s")
    wid = cid * NUM_SUBCORES + sid

    def drain_scatter(b):
        pltpu.make_async_copy(zeros_hbm.at[pl.ds(0, CHUNK)], rows[b],
                              ssems[b]).wait()

    pltpu.sync_copy(zeros_hbm, acc_sh.at[pl.ds(sid * RPT, RPT)])
    plsc.subcore_barrier()

    for hop in range(num_hops):

        def slab_body(s, carry):
            sl = pl.ds(s * SLAB, SLAB)
            pltpu.sync_copy(src_hbm.at[hop, wid, sl], src_v)
            pltpu.sync_copy(dst_hbm.at[hop, wid, sl], dst_v)
            pltpu.sync_copy(w_hbm.at[hop, wid, sl], w_v)
            pltpu.async_copy(x_hbm.at[src_v.at[0]], rows[0], gsems[0])
            pltpu.async_copy(x_hbm.at[src_v.at[1]], rows[1], gsems[1])

            def qbody(q, c2):
                for b in range(NBUF):
                    j = q * NBUF + b
                    b2 = (b + 2) % NBUF
                    pltpu.make_async_copy(x_hbm.at[src_v.at[j]], rows[b],
                                          gsems[b]).wait()
                    if b >= 2:
                        drain_scatter(b2)
                        @pl.when(q < nq - 1)
                        def _():
                            pltpu.async_copy(x_hbm.at[src_v.at[j + 2]],
                                             rows[b2], gsems[b2])
                    else:
                        @pl.when(q > 0)
                        def _():
                            drain_scatter(b2)
                        pltpu.async_copy(x_hbm.at[src_v.at[j + 2]],
                                         rows[b2], gsems[b2])
                    _scale_chunk(w_v, rows[b], j)
                    pltpu.async_copy(rows[b], acc_sh.at[dst_v.at[j]],
                                     ssems[b], add=True)
                return c2

            lax.fori_loop(0, nq, qbody, 0)
            drain_scatter(2)
            drain_scatter(3)
            return carry

        lax.fori_loop(0, num_slabs, slab_body, 0)
        plsc.subcore_barrier()
        pltpu.sync_copy(acc_sh.at[pl.ds(sid * RPT, RPT)],
                        out_hbm.at[hop, cid, pl.ds(sid * RPT, RPT)])
        plsc.subcore_barrier()


def _sc_spmm(x, src, dst, w):
    num_hops, _, num_chunks, _ = src.shape
    zeros = jnp.zeros((RPT, D), jnp.float32)
    mesh = plsc.VectorSubcoreMesh(core_axis_name="c", subcore_axis_name="s")
    f = pl.kernel(
        _sc_body,
        out_type=jax.ShapeDtypeStruct((num_hops, NUM_CORES, N_PAD, D),
                                      jnp.float32),
        mesh=mesh,
        scratch_types=[
            pltpu.VMEM((SLAB, CHUNK), jnp.int32),
            pltpu.VMEM((SLAB, CHUNK), jnp.int32),
            pltpu.VMEM((SLAB, CHUNK), jnp.float32),
            pltpu.VMEM((CHUNK, D), jnp.float32),
            pltpu.VMEM((CHUNK, D), jnp.float32),
            pltpu.VMEM((CHUNK, D), jnp.float32),
            pltpu.VMEM((CHUNK, D), jnp.float32),
            pltpu.VMEM_SHARED((N_PAD, D), jnp.float32),
            pltpu.SemaphoreType.DMA,
            pltpu.SemaphoreType.DMA,
            pltpu.SemaphoreType.DMA,
            pltpu.SemaphoreType.DMA,
            pltpu.SemaphoreType.DMA,
            pltpu.SemaphoreType.DMA,
            pltpu.SemaphoreType.DMA,
            pltpu.SemaphoreType.DMA,
        ],
    )
    return f(x, src, dst, w, zeros)


def _tc_body(p_ref, wih_ref, whh_ref, bih_ref, bhh_ref, g_ref, b_ref, o_ref):
    num_hops = p_ref.shape[0]
    bn = o_ref.shape[0]
    h = jnp.zeros((bn, H), jnp.float32)
    acc = jnp.zeros((bn, H), jnp.float32)
    for c in range(num_hops):
        hx = jnp.maximum(p_ref[c, 0] + p_ref[c, 1], 0.0)
        gi = jnp.dot(hx, wih_ref[...], preferred_element_type=jnp.float32)
        gi = gi + bih_ref[...]
        gh = jnp.dot(h, whh_ref[...], preferred_element_type=jnp.float32)
        gh = gh + bhh_ref[...]
        r = jax.nn.sigmoid(gi[:, :H] + gh[:, :H])
        z = jax.nn.sigmoid(gi[:, H:2 * H] + gh[:, H:2 * H])
        n = jnp.tanh(gi[:, 2 * H:] + r * gh[:, 2 * H:])
        h = (1.0 - z) * n + z * h
        acc = acc + h
    mean = jnp.mean(acc, axis=-1, keepdims=True)
    var = jnp.mean((acc - mean) ** 2, axis=-1, keepdims=True)
    o_ref[...] = (acc - mean) * lax.rsqrt(var + 1e-5) * g_ref[...] + b_ref[...]


def _tc_gru(partials, W_ihT, W_hhT, b_ih, b_hh, gamma, beta, interpret=False):
    num_hops = partials.shape[0]
    bn = 1024
    grid = (N_PAD // bn,)
    return pl.pallas_call(
        _tc_body,
        grid=grid,
        in_specs=[
            pl.BlockSpec((num_hops, NUM_CORES, bn, D),
                         lambda i: (0, 0, i, 0)),
            pl.BlockSpec((D, 3 * H), lambda i: (0, 0)),
            pl.BlockSpec((H, 3 * H), lambda i: (0, 0)),
            pl.BlockSpec((1, 3 * H), lambda i: (0, 0)),
            pl.BlockSpec((1, 3 * H), lambda i: (0, 0)),
            pl.BlockSpec((1, H), lambda i: (0, 0)),
            pl.BlockSpec((1, H), lambda i: (0, 0)),
        ],
        out_specs=pl.BlockSpec((bn, H), lambda i: (i, 0)),
        out_shape=jax.ShapeDtypeStruct((N_PAD, H), jnp.float32),
        interpret=interpret,
    )(partials, W_ihT, W_hhT, b_ih.reshape(1, -1), b_hh.reshape(1, -1),
      gamma.reshape(1, -1), beta.reshape(1, -1))


def kernel(x, edge_weight, W_ih, W_hh, b_ih, b_hh, gamma, beta, edge_index):
    num_hops, _, num_edges = edge_index.shape
    grp = NW * CHUNK * SLAB
    e_pad = ((num_edges + grp - 1) // grp) * grp
    pad = e_pad - num_edges
    pad_idx = jnp.arange(pad, dtype=jnp.int32)
    dst = jnp.concatenate(
        [edge_index[:, 0, :],
         jnp.broadcast_to(pad_idx % N_PAD, (num_hops, pad))], axis=1)
    src = jnp.concatenate(
        [edge_index[:, 1, :],
         jnp.broadcast_to(pad_idx % N_NODES, (num_hops, pad))], axis=1)
    w = jnp.pad(edge_weight, ((0, 0), (0, pad)))
    num_chunks = e_pad // (NW * CHUNK)
    dst = dst.reshape(num_hops, NW, num_chunks, CHUNK)
    src = src.reshape(num_hops, NW, num_chunks, CHUNK)
    w = w.reshape(num_hops, NW, num_chunks, CHUNK)

    partials = _sc_spmm(x, src, dst, w)
    out = _tc_gru(partials, W_ih.T, W_hh.T, b_ih, b_hh, gamma, beta)
    return out[:N_NODES]
